# baseline (speedup 1.0000x reference)
import functools

import jax
import jax.numpy as jnp
from jax import lax
from jax.experimental import pallas as pl
from jax.experimental.pallas import tpu as pltpu
from jax.experimental.pallas import tpu_sc as plsc

N = 50000
E = 800000
V = 10000
D = 64
DH = 32
NCLS = 8
G = 7000

NPAD = 50176
EPAD = 802816
EROWS = 6272
DEG_BINS = 50176
CNT_BINS = 7168

_MESH = plsc.VectorSubcoreMesh(core_axis_name="c", subcore_axis_name="s")
_SC_PARAMS = pltpu.CompilerParams(use_tc_tiling_on_sc=False)
_f32 = jnp.float32
_i32 = jnp.int32


def _zero_vec(ref, n):
    @pl.loop(0, n, step=16)
    def _(i):
        ref.at[pl.ds(i, 16)][...] = jnp.zeros((16,), _f32)


def _sc_hist_gather(dst_hbm, batch_hbm, x_hbm, tab_hbm,
                    deg_hbm, cnt_hbm, e1_hbm,
                    ones_b, ibuf, ib32, xbuf, xb32, rows, zbuf,
                    deg_sh, cnt_sh):
    c = lax.axis_index("c")
    s = lax.axis_index("s")
    w = c * 16 + s

    @pl.loop(0, 128, step=16)
    def _(i):
        ones_b.at[pl.ds(i, 16)][...] = jnp.ones((16,), _f32)
    _zero_vec(zbuf, 3136)

    pltpu.sync_copy(zbuf, deg_sh.at[pl.ds(s * 3136, 3136)])
    pltpu.sync_copy(zbuf.at[pl.ds(0, 448)], cnt_sh.at[pl.ds(s * 448, 448)])
    plsc.subcore_barrier()

    @pl.loop(0, 196)
    def _(ch):
        pltpu.sync_copy(dst_hbm.at[w * 196 + ch], ibuf.at[0])
        pltpu.sync_copy(ones_b, deg_sh.at[ibuf.at[0]], add=True)

    bbase = c * 25088 + s * 1568

    @pl.loop(0, 12)
    def _(ch):
        off = bbase + ch * 128
        pltpu.sync_copy(batch_hbm.at[pl.ds(off, 128)], ibuf.at[0])
        pltpu.sync_copy(ones_b, cnt_sh.at[ibuf.at[0]], add=True)

    pltpu.sync_copy(batch_hbm.at[pl.ds(bbase + 12 * 128, 32)], ib32.at[0])
    pltpu.sync_copy(ones_b.at[pl.ds(0, 32)], cnt_sh.at[ib32.at[0]], add=True)

    xbase = c * 25088 + s * 1568

    @pl.loop(0, 12)
    def _(ch):
        off = xbase + ch * 128
        pltpu.sync_copy(x_hbm.at[pl.ds(off, 128)], xbuf)
        pltpu.sync_copy(tab_hbm.at[xbuf], rows)
        pltpu.sync_copy(rows, e1_hbm.at[pl.ds(off, 128)])

    toff = xbase + 12 * 128
    pltpu.sync_copy(x_hbm.at[pl.ds(toff, 32)], xb32)
    pltpu.sync_copy(tab_hbm.at[xb32], rows.at[pl.ds(0, 32)])
    pltpu.sync_copy(rows.at[pl.ds(0, 32)], e1_hbm.at[pl.ds(toff, 32)])

    plsc.subcore_barrier()
    pltpu.sync_copy(deg_sh.at[pl.ds(s * 3136, 3136)],
                    deg_hbm.at[pl.ds(c * DEG_BINS + s * 3136, 3136)])
    pltpu.sync_copy(cnt_sh.at[pl.ds(s * 448, 448)],
                    cnt_hbm.at[pl.ds(c * CNT_BINS + s * 448, 448)])


def _run_hist_gather(dst2d, batchp, xp, tab):
    k = pl.kernel(
        _sc_hist_gather,
        compiler_params=_SC_PARAMS,
        out_type=[
            jax.ShapeDtypeStruct((2 * DEG_BINS,), _f32),
            jax.ShapeDtypeStruct((2 * CNT_BINS,), _f32),
            jax.ShapeDtypeStruct((NPAD, D), _f32),
        ],
        mesh=_MESH,
        scratch_types=[
            pltpu.VMEM((128,), _f32),
            pltpu.VMEM((1, 128), _i32),
            pltpu.VMEM((1, 32), _i32),
            pltpu.VMEM((128,), _i32),
            pltpu.VMEM((32,), _i32),
            pltpu.VMEM((128, D), _f32),
            pltpu.VMEM((3136,), _f32),
            pltpu.VMEM_SHARED((DEG_BINS,), _f32),
            pltpu.VMEM_SHARED((CNT_BINS,), _f32),
        ],
    )
    return k(dst2d, batchp, xp, tab)


def _sc_edge_agg(g_hbm, src_hbm, dst_hbm, out_hbm,
                 sa, sb, la, lb, rows_a, rows_b, acc_sh,
                 sga, sgb, sla, slb):
    c = lax.axis_index("c")
    s = lax.axis_index("s")
    gref = g_hbm.at[c]

    def load_idx(ch, sbuf, lbuf, sem):
        pltpu.async_copy(src_hbm.at[s * 392 + ch], sbuf.at[0], sem)
        pltpu.async_copy(dst_hbm.at[s * 392 + ch], lbuf.at[0], sem)

    def wait_idx(ch, sbuf, lbuf, sem):
        pltpu.make_async_copy(src_hbm.at[s * 392 + ch], sbuf.at[0], sem).wait()
        pltpu.make_async_copy(dst_hbm.at[s * 392 + ch], lbuf.at[0], sem).wait()

    pltpu.sync_copy(gref.at[pl.ds(s * 3136, 3136)],
                    acc_sh.at[pl.ds(s * 3136, 3136)])
    plsc.subcore_barrier()

    load_idx(0, sa, la, sla)
    wait_idx(0, sa, la, sla)
    pltpu.async_copy(gref.at[sa.at[0]], rows_a, sga)
    load_idx(1, sb, lb, slb)

    @pl.loop(0, 196)
    def _(i):
        ch = 2 * i
        pltpu.make_async_copy(gref.at[sa.at[0]], rows_a, sga).wait()
        wait_idx(ch + 1, sb, lb, slb)
        pltpu.async_copy(gref.at[sb.at[0]], rows_b, sgb)
        pltpu.sync_copy(rows_a, acc_sh.at[la.at[0]], add=True)

        @pl.when(ch + 2 < 392)
        def _():
            load_idx(ch + 2, sa, la, sla)

        pltpu.make_async_copy(gref.at[sb.at[0]], rows_b, sgb).wait()

        @pl.when(ch + 2 < 392)
        def _():
            wait_idx(ch + 2, sa, la, sla)
            pltpu.async_copy(gref.at[sa.at[0]], rows_a, sga)

        pltpu.sync_copy(rows_b, acc_sh.at[lb.at[0]], add=True)

        @pl.when(ch + 3 < 392)
        def _():
            load_idx(ch + 3, sb, lb, slb)

    plsc.subcore_barrier()
    pltpu.sync_copy(acc_sh.at[pl.ds(s * 3136, 3136)],
                    out_hbm.at[c, pl.ds(s * 3136, 3136)])


def _run_edge_agg(g2, src2d, dst2d):
    k = pl.kernel(
        _sc_edge_agg,
        compiler_params=_SC_PARAMS,
        out_type=jax.ShapeDtypeStruct((2, NPAD, DH), _f32),
        mesh=_MESH,
        scratch_types=[
            pltpu.VMEM((1, 128), _i32),
            pltpu.VMEM((1, 128), _i32),
            pltpu.VMEM((1, 128), _i32),
            pltpu.VMEM((1, 128), _i32),
            pltpu.VMEM((128, DH), _f32),
            pltpu.VMEM((128, DH), _f32),
            pltpu.VMEM_SHARED((NPAD, DH), _f32),
            pltpu.SemaphoreType.DMA,
            pltpu.SemaphoreType.DMA,
            pltpu.SemaphoreType.DMA,
            pltpu.SemaphoreType.DMA,
        ],
    )
    return k(g2, src2d, dst2d)


def _sc_pool(h_hbm, batch_hbm, sums_hbm, bbuf, b64, rows, acc_sh):
    c = lax.axis_index("c")
    s = lax.axis_index("s")
    href = h_hbm.at[c]

    @pl.loop(0, 128)
    def _(r):
        @pl.loop(0, DH, step=16)
        def _(j):
            rows.at[r, pl.ds(j, 16)][...] = jnp.zeros((16,), _f32)

    @pl.loop(0, 3)
    def _(q):
        pltpu.sync_copy(rows, acc_sh.at[pl.ds(s * 448 + q * 128, 128)])
    pltpu.sync_copy(rows.at[pl.ds(0, 64)], acc_sh.at[pl.ds(s * 448 + 384, 64)])
    plsc.subcore_barrier()

    rbase = s * 3136

    @pl.loop(0, 24)
    def _(ch):
        off = rbase + ch * 128
        pltpu.sync_copy(href.at[pl.ds(off, 128)], rows)
        pltpu.sync_copy(batch_hbm.at[pl.ds(off, 128)], bbuf.at[0])
        pltpu.sync_copy(rows, acc_sh.at[bbuf.at[0]], add=True)

    toff = rbase + 24 * 128
    pltpu.sync_copy(href.at[pl.ds(toff, 64)], rows.at[pl.ds(0, 64)])
    pltpu.sync_copy(batch_hbm.at[pl.ds(toff, 64)], b64.at[0])
    pltpu.sync_copy(rows.at[pl.ds(0, 64)], acc_sh.at[b64.at[0]], add=True)

    plsc.subcore_barrier()
    pltpu.sync_copy(acc_sh.at[pl.ds(s * 448, 448)],
                    sums_hbm.at[c, pl.ds(s * 448, 448)])


def _run_pool(h2, batchp):
    k = pl.kernel(
        _sc_pool,
        compiler_params=_SC_PARAMS,
        out_type=jax.ShapeDtypeStruct((2, CNT_BINS, DH), _f32),
        mesh=_MESH,
        scratch_types=[
            pltpu.VMEM((1, 128), _i32),
            pltpu.VMEM((1, 64), _i32),
            pltpu.VMEM((128, DH), _f32),
            pltpu.VMEM_SHARED((CNT_BINS, DH), _f32),
        ],
    )
    return k(h2, batchp)


def _tc_prep_body(d0_ref, d1_ref, e1_ref, w1_ref, dinv_ref, g1_ref):
    dv = lax.rsqrt(d0_ref[...] + d1_ref[...] + 1.0)
    dinv_ref[...] = dv
    g = lax.dot_general(e1_ref[...], w1_ref[...], (((1,), (0,)), ((), ())),
                        preferred_element_type=_f32) * dv
    g1_ref[0] = g[:, :DH]
    g1_ref[1] = g[:, DH:]


def _tc_prep(d0, d1, e1, w1):
    return pl.pallas_call(
        _tc_prep_body,
        out_shape=[jax.ShapeDtypeStruct((NPAD, 1), _f32),
                   jax.ShapeDtypeStruct((2, NPAD, DH), _f32)],
        grid=(NPAD // 512,),
        in_specs=[pl.BlockSpec((512, 1), lambda i: (i, 0)),
                  pl.BlockSpec((512, 1), lambda i: (i, 0)),
                  pl.BlockSpec((512, D), lambda i: (i, 0)),
                  pl.BlockSpec((D, D), lambda i: (0, 0))],
        out_specs=[pl.BlockSpec((512, 1), lambda i: (i, 0)),
                   pl.BlockSpec((2, 512, DH), lambda i: (0, i, 0))],
    )(d0, d1, e1, w1)


def _tc_layer_body(agg_ref, dinv_ref, b_ref, w_ref, o_ref):
    dv = dinv_ref[...]
    agg = jnp.concatenate([agg_ref[0], agg_ref[1]], axis=1)
    h = jax.nn.relu(agg * dv + b_ref[...])
    g = lax.dot_general(h, w_ref[...], (((1,), (0,)), ((), ())),
                        preferred_element_type=_f32) * dv
    o_ref[0] = g[:, :DH]
    o_ref[1] = g[:, DH:]


def _tc_layer(agg, dinv, b, w):
    return pl.pallas_call(
        _tc_layer_body,
        out_shape=jax.ShapeDtypeStruct((2, NPAD, DH), _f32),
        grid=(NPAD // 512,),
        in_specs=[pl.BlockSpec((2, 512, DH), lambda i: (0, i, 0)),
                  pl.BlockSpec((512, 1), lambda i: (i, 0)),
                  pl.BlockSpec((D,), lambda i: (0,)),
                  pl.BlockSpec((D, D), lambda i: (0, 0))],
        out_specs=pl.BlockSpec((2, 512, DH), lambda i: (0, i, 0)),
    )(agg, dinv, b, w)


def _tc_act_body(agg_ref, dinv_ref, b_ref, o_ref):
    agg = jnp.concatenate([agg_ref[0], agg_ref[1]], axis=1)
    h = jax.nn.relu(agg * dinv_ref[...] + b_ref[...])
    o_ref[0] = h[:, :DH]
    o_ref[1] = h[:, DH:]


def _tc_act(agg, dinv, b):
    return pl.pallas_call(
        _tc_act_body,
        out_shape=jax.ShapeDtypeStruct((2, NPAD, DH), _f32),
        grid=(NPAD // 512,),
        in_specs=[pl.BlockSpec((2, 512, DH), lambda i: (0, i, 0)),
                  pl.BlockSpec((512, 1), lambda i: (i, 0)),
                  pl.BlockSpec((D,), lambda i: (0,))],
        out_specs=pl.BlockSpec((2, 512, DH), lambda i: (0, i, 0)),
    )(agg, dinv, b)


def _tc_head_body(s_ref, c0_ref, c1_ref, w_ref, b_ref, o_ref):
    cnt = jnp.maximum(c0_ref[...] + c1_ref[...], 1.0)
    pooled = jnp.concatenate([s_ref[0], s_ref[1]], axis=1) / cnt
    o_ref[...] = lax.dot_general(pooled, w_ref[...], (((1,), (0,)), ((), ())),
                                 preferred_element_type=_f32) + b_ref[...]


def _tc_head(sums, c0, c1, wlin, blin):
    return pl.pallas_call(
        _tc_head_body,
        out_shape=jax.ShapeDtypeStruct((G, NCLS), _f32),
        grid=(G // 1000,),
        in_specs=[pl.BlockSpec((2, 1000, DH), lambda i: (0, i, 0)),
                  pl.BlockSpec((1000, 1), lambda i: (i, 0)),
                  pl.BlockSpec((1000, 1), lambda i: (i, 0)),
                  pl.BlockSpec((D, NCLS), lambda i: (0, 0)),
                  pl.BlockSpec((NCLS,), lambda i: (0,))],
        out_specs=pl.BlockSpec((1000, NCLS), lambda i: (i, 0)),
    )(sums, c0, c1, wlin, blin)


def kernel(x, edge_index, batch, emb, W1, b1, W2, b2, Wlin, blin):
    src = edge_index[0].astype(_i32)
    dst = edge_index[1].astype(_i32)
    src2d = jnp.concatenate([src, jnp.zeros((EPAD - E,), _i32)]).reshape(EROWS, 128)
    dst2d = jnp.concatenate([dst, jnp.full((EPAD - E,), N, _i32)]).reshape(EROWS, 128)
    batchp = jnp.concatenate([batch.astype(_i32), jnp.full((NPAD - N,), G, _i32)])
    xp = jnp.concatenate([x.astype(_i32), jnp.zeros((NPAD - N,), _i32)])

    deg_p, cnt_p, e1 = _run_hist_gather(dst2d, batchp, xp, emb)

    d0 = deg_p[:DEG_BINS, None]
    d1 = deg_p[DEG_BINS:, None]
    dinv, g1 = _tc_prep(d0, d1, e1, W1)

    agg1 = _run_edge_agg(g1, src2d, dst2d)
    g2 = _tc_layer(agg1, dinv, b1, W2)
    agg2 = _run_edge_agg(g2, src2d, dst2d)
    h2 = _tc_act(agg2, dinv, b2)

    sums = _run_pool(h2, batchp)
    c0 = cnt_p[:G, None]
    c1 = cnt_p[CNT_BINS:CNT_BINS + G, None]
    return _tc_head(sums[:, :G], c0, c1, Wlin, blin)


# speedup vs baseline: 1.1731x; 1.1731x over previous
import functools

import jax
import jax.numpy as jnp
from jax import lax
from jax.experimental import pallas as pl
from jax.experimental.pallas import tpu as pltpu
from jax.experimental.pallas import tpu_sc as plsc

N = 50000
E = 800000
V = 10000
D = 64
DH = 32
NCLS = 8
G = 7000

NPAD = 50176
EPAD = 802816
EROWS = 6272
DEG_BINS = 50176
CNT_BINS = 7168

_MESH = plsc.VectorSubcoreMesh(core_axis_name="c", subcore_axis_name="s")
_SC_PARAMS = pltpu.CompilerParams(use_tc_tiling_on_sc=False)
_f32 = jnp.float32
_i32 = jnp.int32


def _zero_vec(ref, n):
    @pl.loop(0, n, step=16)
    def _(i):
        ref.at[pl.ds(i, 16)][...] = jnp.zeros((16,), _f32)


def _sc_hist_gather(dst_hbm, batch_hbm, x_hbm, tab_hbm,
                    deg_hbm, cnt_hbm, e1_hbm,
                    ones_b, ibuf, ib32, xbuf, xb32, rows, zbuf,
                    deg_sh, cnt_sh):
    c = lax.axis_index("c")
    s = lax.axis_index("s")
    w = c * 16 + s

    @pl.loop(0, 128, step=16)
    def _(i):
        ones_b.at[pl.ds(i, 16)][...] = jnp.ones((16,), _f32)
    _zero_vec(zbuf, 3136)

    pltpu.sync_copy(zbuf, deg_sh.at[pl.ds(s * 3136, 3136)])
    pltpu.sync_copy(zbuf.at[pl.ds(0, 448)], cnt_sh.at[pl.ds(s * 448, 448)])
    plsc.subcore_barrier()

    @pl.loop(0, 196)
    def _(ch):
        pltpu.sync_copy(dst_hbm.at[w * 196 + ch], ibuf.at[0])
        pltpu.sync_copy(ones_b, deg_sh.at[ibuf.at[0]], add=True)

    bbase = c * 25088 + s * 1568

    @pl.loop(0, 12)
    def _(ch):
        off = bbase + ch * 128
        pltpu.sync_copy(batch_hbm.at[pl.ds(off, 128)], ibuf.at[0])
        pltpu.sync_copy(ones_b, cnt_sh.at[ibuf.at[0]], add=True)

    pltpu.sync_copy(batch_hbm.at[pl.ds(bbase + 12 * 128, 32)], ib32.at[0])
    pltpu.sync_copy(ones_b.at[pl.ds(0, 32)], cnt_sh.at[ib32.at[0]], add=True)

    xbase = c * 25088 + s * 1568

    @pl.loop(0, 12)
    def _(ch):
        off = xbase + ch * 128
        pltpu.sync_copy(x_hbm.at[pl.ds(off, 128)], xbuf)
        pltpu.sync_copy(tab_hbm.at[xbuf], rows)
        pltpu.sync_copy(rows, e1_hbm.at[pl.ds(off, 128)])

    toff = xbase + 12 * 128
    pltpu.sync_copy(x_hbm.at[pl.ds(toff, 32)], xb32)
    pltpu.sync_copy(tab_hbm.at[xb32], rows.at[pl.ds(0, 32)])
    pltpu.sync_copy(rows.at[pl.ds(0, 32)], e1_hbm.at[pl.ds(toff, 32)])

    plsc.subcore_barrier()
    pltpu.sync_copy(deg_sh.at[pl.ds(s * 3136, 3136)],
                    deg_hbm.at[pl.ds(c * DEG_BINS + s * 3136, 3136)])
    pltpu.sync_copy(cnt_sh.at[pl.ds(s * 448, 448)],
                    cnt_hbm.at[pl.ds(c * CNT_BINS + s * 448, 448)])


def _run_hist_gather(dst2d, batchp, xp, tab):
    k = pl.kernel(
        _sc_hist_gather,
        compiler_params=_SC_PARAMS,
        out_type=[
            jax.ShapeDtypeStruct((2 * DEG_BINS,), _f32),
            jax.ShapeDtypeStruct((2 * CNT_BINS,), _f32),
            jax.ShapeDtypeStruct((NPAD, D), _f32),
        ],
        mesh=_MESH,
        scratch_types=[
            pltpu.VMEM((128,), _f32),
            pltpu.VMEM((1, 128), _i32),
            pltpu.VMEM((1, 32), _i32),
            pltpu.VMEM((128,), _i32),
            pltpu.VMEM((32,), _i32),
            pltpu.VMEM((128, D), _f32),
            pltpu.VMEM((3136,), _f32),
            pltpu.VMEM_SHARED((DEG_BINS,), _f32),
            pltpu.VMEM_SHARED((CNT_BINS,), _f32),
        ],
    )
    return k(dst2d, batchp, xp, tab)


def _sc_edge_agg(g_hbm, src_hbm, dst_hbm, out_hbm,
                 sa, sb, la, lb, rows_a, rows_b, acc_sh,
                 sga, sgb, sla, slb):
    c = lax.axis_index("c")
    s = lax.axis_index("s")
    gref = g_hbm.at[c]
    base = s * 50176

    def load_idx(ch, sbuf, lbuf, sem):
        off = base + ch * 256
        pltpu.async_copy(src_hbm.at[pl.ds(off, 256)], sbuf, sem)
        pltpu.async_copy(dst_hbm.at[pl.ds(off, 256)], lbuf, sem)

    def wait_idx(ch, sbuf, lbuf, sem):
        off = base + ch * 256
        pltpu.make_async_copy(src_hbm.at[pl.ds(off, 256)], sbuf, sem).wait()
        pltpu.make_async_copy(dst_hbm.at[pl.ds(off, 256)], lbuf, sem).wait()

    pltpu.sync_copy(gref.at[pl.ds(s * 3136, 3136)],
                    acc_sh.at[pl.ds(s * 3136, 3136)])
    plsc.subcore_barrier()

    load_idx(0, sa, la, sla)
    wait_idx(0, sa, la, sla)
    pltpu.async_copy(gref.at[sa], rows_a, sga)
    load_idx(1, sb, lb, slb)

    @pl.loop(0, 98)
    def _(i):
        ch = 2 * i
        pltpu.make_async_copy(gref.at[sa], rows_a, sga).wait()
        wait_idx(ch + 1, sb, lb, slb)
        pltpu.async_copy(gref.at[sb], rows_b, sgb)
        pltpu.sync_copy(rows_a, acc_sh.at[la], add=True)

        @pl.when(ch + 2 < 196)
        def _():
            load_idx(ch + 2, sa, la, sla)

        pltpu.make_async_copy(gref.at[sb], rows_b, sgb).wait()

        @pl.when(ch + 2 < 196)
        def _():
            wait_idx(ch + 2, sa, la, sla)
            pltpu.async_copy(gref.at[sa], rows_a, sga)

        pltpu.sync_copy(rows_b, acc_sh.at[lb], add=True)

        @pl.when(ch + 3 < 196)
        def _():
            load_idx(ch + 3, sb, lb, slb)

    plsc.subcore_barrier()
    pltpu.sync_copy(acc_sh.at[pl.ds(s * 3136, 3136)],
                    out_hbm.at[c, pl.ds(s * 3136, 3136)])


def _run_edge_agg(g2, src1, dst1):
    k = pl.kernel(
        _sc_edge_agg,
        compiler_params=_SC_PARAMS,
        out_type=jax.ShapeDtypeStruct((2, NPAD, DH), _f32),
        mesh=_MESH,
        scratch_types=[
            pltpu.VMEM((256,), _i32),
            pltpu.VMEM((256,), _i32),
            pltpu.VMEM((256,), _i32),
            pltpu.VMEM((256,), _i32),
            pltpu.VMEM((256, DH), _f32),
            pltpu.VMEM((256, DH), _f32),
            pltpu.VMEM_SHARED((NPAD, DH), _f32),
            pltpu.SemaphoreType.DMA,
            pltpu.SemaphoreType.DMA,
            pltpu.SemaphoreType.DMA,
            pltpu.SemaphoreType.DMA,
        ],
    )
    return k(g2, src1, dst1)


def _sc_pool(h_hbm, batch_hbm, sums_hbm, bbuf, b64, rows, acc_sh):
    c = lax.axis_index("c")
    s = lax.axis_index("s")
    href = h_hbm.at[c]

    @pl.loop(0, 128)
    def _(r):
        @pl.loop(0, DH, step=16)
        def _(j):
            rows.at[r, pl.ds(j, 16)][...] = jnp.zeros((16,), _f32)

    @pl.loop(0, 3)
    def _(q):
        pltpu.sync_copy(rows, acc_sh.at[pl.ds(s * 448 + q * 128, 128)])
    pltpu.sync_copy(rows.at[pl.ds(0, 64)], acc_sh.at[pl.ds(s * 448 + 384, 64)])
    plsc.subcore_barrier()

    rbase = s * 3136

    @pl.loop(0, 24)
    def _(ch):
        off = rbase + ch * 128
        pltpu.sync_copy(href.at[pl.ds(off, 128)], rows)
        pltpu.sync_copy(batch_hbm.at[pl.ds(off, 128)], bbuf.at[0])
        pltpu.sync_copy(rows, acc_sh.at[bbuf.at[0]], add=True)

    toff = rbase + 24 * 128
    pltpu.sync_copy(href.at[pl.ds(toff, 64)], rows.at[pl.ds(0, 64)])
    pltpu.sync_copy(batch_hbm.at[pl.ds(toff, 64)], b64.at[0])
    pltpu.sync_copy(rows.at[pl.ds(0, 64)], acc_sh.at[b64.at[0]], add=True)

    plsc.subcore_barrier()
    pltpu.sync_copy(acc_sh.at[pl.ds(s * 448, 448)],
                    sums_hbm.at[c, pl.ds(s * 448, 448)])


def _run_pool(h2, batchp):
    k = pl.kernel(
        _sc_pool,
        compiler_params=_SC_PARAMS,
        out_type=jax.ShapeDtypeStruct((2, CNT_BINS, DH), _f32),
        mesh=_MESH,
        scratch_types=[
            pltpu.VMEM((1, 128), _i32),
            pltpu.VMEM((1, 64), _i32),
            pltpu.VMEM((128, DH), _f32),
            pltpu.VMEM_SHARED((CNT_BINS, DH), _f32),
        ],
    )
    return k(h2, batchp)


def _tc_prep_body(d0_ref, d1_ref, e1_ref, w1_ref, dinv_ref, g1_ref):
    dv = lax.rsqrt(d0_ref[...] + d1_ref[...] + 1.0)
    dinv_ref[...] = dv
    g = lax.dot_general(e1_ref[...], w1_ref[...], (((1,), (0,)), ((), ())),
                        preferred_element_type=_f32) * dv
    g1_ref[0] = g[:, :DH]
    g1_ref[1] = g[:, DH:]


def _tc_prep(d0, d1, e1, w1):
    return pl.pallas_call(
        _tc_prep_body,
        out_shape=[jax.ShapeDtypeStruct((NPAD, 1), _f32),
                   jax.ShapeDtypeStruct((2, NPAD, DH), _f32)],
        grid=(NPAD // 512,),
        in_specs=[pl.BlockSpec((512, 1), lambda i: (i, 0)),
                  pl.BlockSpec((512, 1), lambda i: (i, 0)),
                  pl.BlockSpec((512, D), lambda i: (i, 0)),
                  pl.BlockSpec((D, D), lambda i: (0, 0))],
        out_specs=[pl.BlockSpec((512, 1), lambda i: (i, 0)),
                   pl.BlockSpec((2, 512, DH), lambda i: (0, i, 0))],
    )(d0, d1, e1, w1)


def _tc_layer_body(agg_ref, dinv_ref, b_ref, w_ref, o_ref):
    dv = dinv_ref[...]
    agg = jnp.concatenate([agg_ref[0], agg_ref[1]], axis=1)
    h = jax.nn.relu(agg * dv + b_ref[...])
    g = lax.dot_general(h, w_ref[...], (((1,), (0,)), ((), ())),
                        preferred_element_type=_f32) * dv
    o_ref[0] = g[:, :DH]
    o_ref[1] = g[:, DH:]


def _tc_layer(agg, dinv, b, w):
    return pl.pallas_call(
        _tc_layer_body,
        out_shape=jax.ShapeDtypeStruct((2, NPAD, DH), _f32),
        grid=(NPAD // 512,),
        in_specs=[pl.BlockSpec((2, 512, DH), lambda i: (0, i, 0)),
                  pl.BlockSpec((512, 1), lambda i: (i, 0)),
                  pl.BlockSpec((D,), lambda i: (0,)),
                  pl.BlockSpec((D, D), lambda i: (0, 0))],
        out_specs=pl.BlockSpec((2, 512, DH), lambda i: (0, i, 0)),
    )(agg, dinv, b, w)


def _tc_act_body(agg_ref, dinv_ref, b_ref, o_ref):
    agg = jnp.concatenate([agg_ref[0], agg_ref[1]], axis=1)
    h = jax.nn.relu(agg * dinv_ref[...] + b_ref[...])
    o_ref[0] = h[:, :DH]
    o_ref[1] = h[:, DH:]


def _tc_act(agg, dinv, b):
    return pl.pallas_call(
        _tc_act_body,
        out_shape=jax.ShapeDtypeStruct((2, NPAD, DH), _f32),
        grid=(NPAD // 512,),
        in_specs=[pl.BlockSpec((2, 512, DH), lambda i: (0, i, 0)),
                  pl.BlockSpec((512, 1), lambda i: (i, 0)),
                  pl.BlockSpec((D,), lambda i: (0,))],
        out_specs=pl.BlockSpec((2, 512, DH), lambda i: (0, i, 0)),
    )(agg, dinv, b)


def _tc_head_body(s_ref, c0_ref, c1_ref, w_ref, b_ref, o_ref):
    cnt = jnp.maximum(c0_ref[...] + c1_ref[...], 1.0)
    pooled = jnp.concatenate([s_ref[0], s_ref[1]], axis=1) / cnt
    o_ref[...] = lax.dot_general(pooled, w_ref[...], (((1,), (0,)), ((), ())),
                                 preferred_element_type=_f32) + b_ref[...]


def _tc_head(sums, c0, c1, wlin, blin):
    return pl.pallas_call(
        _tc_head_body,
        out_shape=jax.ShapeDtypeStruct((G, NCLS), _f32),
        grid=(G // 1000,),
        in_specs=[pl.BlockSpec((2, 1000, DH), lambda i: (0, i, 0)),
                  pl.BlockSpec((1000, 1), lambda i: (i, 0)),
                  pl.BlockSpec((1000, 1), lambda i: (i, 0)),
                  pl.BlockSpec((D, NCLS), lambda i: (0, 0)),
                  pl.BlockSpec((NCLS,), lambda i: (0,))],
        out_specs=pl.BlockSpec((1000, NCLS), lambda i: (i, 0)),
    )(sums, c0, c1, wlin, blin)


def kernel(x, edge_index, batch, emb, W1, b1, W2, b2, Wlin, blin):
    src = edge_index[0].astype(_i32)
    dst = edge_index[1].astype(_i32)
    src1 = jnp.concatenate([src, jnp.zeros((EPAD - E,), _i32)])
    dst1 = jnp.concatenate([dst, jnp.full((EPAD - E,), N, _i32)])
    dst2d = dst1.reshape(EROWS, 128)
    batchp = jnp.concatenate([batch.astype(_i32), jnp.full((NPAD - N,), G, _i32)])
    xp = jnp.concatenate([x.astype(_i32), jnp.zeros((NPAD - N,), _i32)])

    deg_p, cnt_p, e1 = _run_hist_gather(dst2d, batchp, xp, emb)

    d0 = deg_p[:DEG_BINS, None]
    d1 = deg_p[DEG_BINS:, None]
    dinv, g1 = _tc_prep(d0, d1, e1, W1)

    agg1 = _run_edge_agg(g1, src1, dst1)
    g2 = _tc_layer(agg1, dinv, b1, W2)
    agg2 = _run_edge_agg(g2, src1, dst1)
    h2 = _tc_act(agg2, dinv, b2)

    sums = _run_pool(h2, batchp)
    c0 = cnt_p[:G, None]
    c1 = cnt_p[CNT_BINS:CNT_BINS + G, None]
    return _tc_head(sums[:, :G], c0, c1, Wlin, blin)


# speedup vs baseline: 1.1731x; 1.0000x over previous
import jax
import jax.numpy as jnp
from jax import lax
from jax.experimental import pallas as pl
from jax.experimental.pallas import tpu as pltpu
from jax.experimental.pallas import tpu_sc as plsc

N = 50000
E = 800000
V = 10000
D = 64
DH = 32
NCLS = 8
G = 7000

NPAD = 50176
EPAD = 802816
EROWS = 6272
DEG_BINS = 50176
CNT_BINS = 7168

_MESH = plsc.VectorSubcoreMesh(core_axis_name="c", subcore_axis_name="s")
_SC_PARAMS = pltpu.CompilerParams(use_tc_tiling_on_sc=False)
_f32 = jnp.float32
_i32 = jnp.int32


def _zero_vec(ref, n):
    @pl.loop(0, n, step=16)
    def _(i):
        ref.at[pl.ds(i, 16)][...] = jnp.zeros((16,), _f32)


def _sc_hist_gather(dst_hbm, batch_hbm, x_hbm, tab_hbm,
                    deg_hbm, cnt_hbm, e1_hbm,
                    ones_b, ibuf, ib32, xbuf, xb32, rows, zbuf,
                    deg_sh, cnt_sh):
    c = lax.axis_index("c")
    s = lax.axis_index("s")
    w = c * 16 + s

    @pl.loop(0, 128, step=16)
    def _(i):
        ones_b.at[pl.ds(i, 16)][...] = jnp.ones((16,), _f32)
    _zero_vec(zbuf, 3136)

    pltpu.sync_copy(zbuf, deg_sh.at[pl.ds(s * 3136, 3136)])
    pltpu.sync_copy(zbuf.at[pl.ds(0, 448)], cnt_sh.at[pl.ds(s * 448, 448)])
    plsc.subcore_barrier()

    @pl.loop(0, 196)
    def _(ch):
        pltpu.sync_copy(dst_hbm.at[w * 196 + ch], ibuf.at[0])
        pltpu.sync_copy(ones_b, deg_sh.at[ibuf.at[0]], add=True)

    bbase = c * 25088 + s * 1568

    @pl.loop(0, 12)
    def _(ch):
        off = bbase + ch * 128
        pltpu.sync_copy(batch_hbm.at[pl.ds(off, 128)], ibuf.at[0])
        pltpu.sync_copy(ones_b, cnt_sh.at[ibuf.at[0]], add=True)

    pltpu.sync_copy(batch_hbm.at[pl.ds(bbase + 12 * 128, 32)], ib32.at[0])
    pltpu.sync_copy(ones_b.at[pl.ds(0, 32)], cnt_sh.at[ib32.at[0]], add=True)

    xbase = c * 25088 + s * 1568

    @pl.loop(0, 12)
    def _(ch):
        off = xbase + ch * 128
        pltpu.sync_copy(x_hbm.at[pl.ds(off, 128)], xbuf)
        pltpu.sync_copy(tab_hbm.at[xbuf], rows)
        pltpu.sync_copy(rows, e1_hbm.at[pl.ds(off, 128)])

    toff = xbase + 12 * 128
    pltpu.sync_copy(x_hbm.at[pl.ds(toff, 32)], xb32)
    pltpu.sync_copy(tab_hbm.at[xb32], rows.at[pl.ds(0, 32)])
    pltpu.sync_copy(rows.at[pl.ds(0, 32)], e1_hbm.at[pl.ds(toff, 32)])

    plsc.subcore_barrier()
    pltpu.sync_copy(deg_sh.at[pl.ds(s * 3136, 3136)],
                    deg_hbm.at[pl.ds(c * DEG_BINS + s * 3136, 3136)])
    pltpu.sync_copy(cnt_sh.at[pl.ds(s * 448, 448)],
                    cnt_hbm.at[pl.ds(c * CNT_BINS + s * 448, 448)])


def _run_hist_gather(dst2d, batchp, xp, tab):
    k = pl.kernel(
        _sc_hist_gather,
        compiler_params=_SC_PARAMS,
        out_type=[
            jax.ShapeDtypeStruct((2 * DEG_BINS,), _f32),
            jax.ShapeDtypeStruct((2 * CNT_BINS,), _f32),
            jax.ShapeDtypeStruct((NPAD, D), _f32),
        ],
        mesh=_MESH,
        scratch_types=[
            pltpu.VMEM((128,), _f32),
            pltpu.VMEM((1, 128), _i32),
            pltpu.VMEM((1, 32), _i32),
            pltpu.VMEM((128,), _i32),
            pltpu.VMEM((32,), _i32),
            pltpu.VMEM((128, D), _f32),
            pltpu.VMEM((3136,), _f32),
            pltpu.VMEM_SHARED((DEG_BINS,), _f32),
            pltpu.VMEM_SHARED((CNT_BINS,), _f32),
        ],
    )
    return k(dst2d, batchp, xp, tab)


def _sc_edge_agg(g_hbm, src_hbm, dst_hbm, out_hbm,
                 sa, sb, la, lb, rows_a, rows_b, acc_sh,
                 sga, sgb, sla, slb):
    c = lax.axis_index("c")
    s = lax.axis_index("s")
    gref = g_hbm.at[c]
    base = s * 50176

    def load_idx(ch, sbuf, lbuf, sem):
        off = base + ch * 256
        pltpu.async_copy(src_hbm.at[pl.ds(off, 256)], sbuf, sem)
        pltpu.async_copy(dst_hbm.at[pl.ds(off, 256)], lbuf, sem)

    def wait_idx(ch, sbuf, lbuf, sem):
        off = base + ch * 256
        pltpu.make_async_copy(src_hbm.at[pl.ds(off, 256)], sbuf, sem).wait()
        pltpu.make_async_copy(dst_hbm.at[pl.ds(off, 256)], lbuf, sem).wait()

    pltpu.sync_copy(gref.at[pl.ds(s * 3136, 3136)],
                    acc_sh.at[pl.ds(s * 3136, 3136)])
    plsc.subcore_barrier()

    load_idx(0, sa, la, sla)
    wait_idx(0, sa, la, sla)
    pltpu.async_copy(gref.at[sa], rows_a, sga)
    load_idx(1, sb, lb, slb)

    @pl.loop(0, 98)
    def _(i):
        ch = 2 * i
        pltpu.make_async_copy(gref.at[sa], rows_a, sga).wait()
        wait_idx(ch + 1, sb, lb, slb)
        pltpu.async_copy(gref.at[sb], rows_b, sgb)
        pltpu.sync_copy(rows_a, acc_sh.at[la], add=True)

        @pl.when(ch + 2 < 196)
        def _():
            load_idx(ch + 2, sa, la, sla)

        pltpu.make_async_copy(gref.at[sb], rows_b, sgb).wait()

        @pl.when(ch + 2 < 196)
        def _():
            wait_idx(ch + 2, sa, la, sla)
            pltpu.async_copy(gref.at[sa], rows_a, sga)

        pltpu.sync_copy(rows_b, acc_sh.at[lb], add=True)

        @pl.when(ch + 3 < 196)
        def _():
            load_idx(ch + 3, sb, lb, slb)

    plsc.subcore_barrier()
    pltpu.sync_copy(acc_sh.at[pl.ds(s * 3136, 3136)],
                    out_hbm.at[c, pl.ds(s * 3136, 3136)])


def _run_edge_agg(g2, src1, dst1):
    k = pl.kernel(
        _sc_edge_agg,
        compiler_params=_SC_PARAMS,
        out_type=jax.ShapeDtypeStruct((2, NPAD, DH), _f32),
        mesh=_MESH,
        scratch_types=[
            pltpu.VMEM((256,), _i32),
            pltpu.VMEM((256,), _i32),
            pltpu.VMEM((256,), _i32),
            pltpu.VMEM((256,), _i32),
            pltpu.VMEM((256, DH), _f32),
            pltpu.VMEM((256, DH), _f32),
            pltpu.VMEM_SHARED((NPAD, DH), _f32),
            pltpu.SemaphoreType.DMA,
            pltpu.SemaphoreType.DMA,
            pltpu.SemaphoreType.DMA,
            pltpu.SemaphoreType.DMA,
        ],
    )
    return k(g2, src1, dst1)


def _sc_pool(h_hbm, batch_hbm, sums_hbm, bbuf, b64, rows, acc_sh):
    c = lax.axis_index("c")
    s = lax.axis_index("s")
    href = h_hbm.at[c]

    @pl.loop(0, 128)
    def _(r):
        @pl.loop(0, DH, step=16)
        def _(j):
            rows.at[r, pl.ds(j, 16)][...] = jnp.zeros((16,), _f32)

    @pl.loop(0, 3)
    def _(q):
        pltpu.sync_copy(rows, acc_sh.at[pl.ds(s * 448 + q * 128, 128)])
    pltpu.sync_copy(rows.at[pl.ds(0, 64)], acc_sh.at[pl.ds(s * 448 + 384, 64)])
    plsc.subcore_barrier()

    rbase = s * 3136

    @pl.loop(0, 24)
    def _(ch):
        off = rbase + ch * 128
        pltpu.sync_copy(href.at[pl.ds(off, 128)], rows)
        pltpu.sync_copy(batch_hbm.at[pl.ds(off, 128)], bbuf.at[0])
        pltpu.sync_copy(rows, acc_sh.at[bbuf.at[0]], add=True)

    toff = rbase + 24 * 128
    pltpu.sync_copy(href.at[pl.ds(toff, 64)], rows.at[pl.ds(0, 64)])
    pltpu.sync_copy(batch_hbm.at[pl.ds(toff, 64)], b64.at[0])
    pltpu.sync_copy(rows.at[pl.ds(0, 64)], acc_sh.at[b64.at[0]], add=True)

    plsc.subcore_barrier()
    pltpu.sync_copy(acc_sh.at[pl.ds(s * 448, 448)],
                    sums_hbm.at[c, pl.ds(s * 448, 448)])


def _run_pool(h2, batchp):
    k = pl.kernel(
        _sc_pool,
        compiler_params=_SC_PARAMS,
        out_type=jax.ShapeDtypeStruct((2, CNT_BINS, DH), _f32),
        mesh=_MESH,
        scratch_types=[
            pltpu.VMEM((1, 128), _i32),
            pltpu.VMEM((1, 64), _i32),
            pltpu.VMEM((128, DH), _f32),
            pltpu.VMEM_SHARED((CNT_BINS, DH), _f32),
        ],
    )
    return k(h2, batchp)


def _tc_prep_body(d0_ref, d1_ref, e1_ref, w1_ref, dinv_ref, g1_ref):
    dv = lax.rsqrt(d0_ref[...] + d1_ref[...] + 1.0)
    dinv_ref[...] = dv
    g = lax.dot_general(e1_ref[...], w1_ref[...], (((1,), (0,)), ((), ())),
                        preferred_element_type=_f32) * dv
    g1_ref[0] = g[:, :DH]
    g1_ref[1] = g[:, DH:]


def _tc_prep(d0, d1, e1, w1):
    return pl.pallas_call(
        _tc_prep_body,
        out_shape=[jax.ShapeDtypeStruct((NPAD, 1), _f32),
                   jax.ShapeDtypeStruct((2, NPAD, DH), _f32)],
        grid=(NPAD // 512,),
        in_specs=[pl.BlockSpec((512, 1), lambda i: (i, 0)),
                  pl.BlockSpec((512, 1), lambda i: (i, 0)),
                  pl.BlockSpec((512, D), lambda i: (i, 0)),
                  pl.BlockSpec((D, D), lambda i: (0, 0))],
        out_specs=[pl.BlockSpec((512, 1), lambda i: (i, 0)),
                   pl.BlockSpec((2, 512, DH), lambda i: (0, i, 0))],
    )(d0, d1, e1, w1)


def _tc_layer_body(agg_ref, dinv_ref, b_ref, w_ref, o_ref):
    dv = dinv_ref[...]
    agg = jnp.concatenate([agg_ref[0], agg_ref[1]], axis=1)
    h = jax.nn.relu(agg * dv + b_ref[...])
    g = lax.dot_general(h, w_ref[...], (((1,), (0,)), ((), ())),
                        preferred_element_type=_f32) * dv
    o_ref[0] = g[:, :DH]
    o_ref[1] = g[:, DH:]


def _tc_layer(agg, dinv, b, w):
    return pl.pallas_call(
        _tc_layer_body,
        out_shape=jax.ShapeDtypeStruct((2, NPAD, DH), _f32),
        grid=(NPAD // 512,),
        in_specs=[pl.BlockSpec((2, 512, DH), lambda i: (0, i, 0)),
                  pl.BlockSpec((512, 1), lambda i: (i, 0)),
                  pl.BlockSpec((D,), lambda i: (0,)),
                  pl.BlockSpec((D, D), lambda i: (0, 0))],
        out_specs=pl.BlockSpec((2, 512, DH), lambda i: (0, i, 0)),
    )(agg, dinv, b, w)


def _tc_act_body(agg_ref, dinv_ref, b_ref, o_ref):
    agg = jnp.concatenate([agg_ref[0], agg_ref[1]], axis=1)
    h = jax.nn.relu(agg * dinv_ref[...] + b_ref[...])
    o_ref[0] = h[:, :DH]
    o_ref[1] = h[:, DH:]


def _tc_act(agg, dinv, b):
    return pl.pallas_call(
        _tc_act_body,
        out_shape=jax.ShapeDtypeStruct((2, NPAD, DH), _f32),
        grid=(NPAD // 512,),
        in_specs=[pl.BlockSpec((2, 512, DH), lambda i: (0, i, 0)),
                  pl.BlockSpec((512, 1), lambda i: (i, 0)),
                  pl.BlockSpec((D,), lambda i: (0,))],
        out_specs=pl.BlockSpec((2, 512, DH), lambda i: (0, i, 0)),
    )(agg, dinv, b)


def _tc_head_body(s_ref, c0_ref, c1_ref, w_ref, b_ref, o_ref):
    cnt = jnp.maximum(c0_ref[...] + c1_ref[...], 1.0)
    pooled = jnp.concatenate([s_ref[0], s_ref[1]], axis=1) / cnt
    o_ref[...] = lax.dot_general(pooled, w_ref[...], (((1,), (0,)), ((), ())),
                                 preferred_element_type=_f32) + b_ref[...]


def _tc_head(sums, c0, c1, wlin, blin):
    return pl.pallas_call(
        _tc_head_body,
        out_shape=jax.ShapeDtypeStruct((G, NCLS), _f32),
        grid=(G // 1000,),
        in_specs=[pl.BlockSpec((2, 1000, DH), lambda i: (0, i, 0)),
                  pl.BlockSpec((1000, 1), lambda i: (i, 0)),
                  pl.BlockSpec((1000, 1), lambda i: (i, 0)),
                  pl.BlockSpec((D, NCLS), lambda i: (0, 0)),
                  pl.BlockSpec((NCLS,), lambda i: (0,))],
        out_specs=pl.BlockSpec((1000, NCLS), lambda i: (i, 0)),
    )(sums, c0, c1, wlin, blin)


def kernel(x, edge_index, batch, emb, W1, b1, W2, b2, Wlin, blin):
    src = edge_index[0].astype(_i32)
    dst = edge_index[1].astype(_i32)
    src1 = jnp.concatenate([src, jnp.zeros((EPAD - E,), _i32)])
    dst1 = jnp.concatenate([dst, jnp.full((EPAD - E,), N, _i32)])
    dst2d = dst1.reshape(EROWS, 128)
    batchp = jnp.concatenate([batch.astype(_i32), jnp.full((NPAD - N,), G, _i32)])
    xp = jnp.concatenate([x.astype(_i32), jnp.zeros((NPAD - N,), _i32)])

    deg_p, cnt_p, e1 = _run_hist_gather(dst2d, batchp, xp, emb)

    d0 = deg_p[:DEG_BINS, None]
    d1 = deg_p[DEG_BINS:, None]
    dinv, g1 = _tc_prep(d0, d1, e1, W1)

    agg1 = _run_edge_agg(g1, src1, dst1)
    g2 = _tc_layer(agg1, dinv, b1, W2)
    agg2 = _run_edge_agg(g2, src1, dst1)
    h2 = _tc_act(agg2, dinv, b2)

    sums = _run_pool(h2, batchp)
    c0 = cnt_p[:G, None]
    c1 = cnt_p[CNT_BINS:CNT_BINS + G, None]
    return _tc_head(sums[:, :G], c0, c1, Wlin, blin)
